# Initial kernel scaffold; baseline (speedup 1.0000x reference)
import jax
import jax.numpy as jnp
from jax.experimental import pallas as pl


def kernel(values):
    raise NotImplementedError("write your pallas kernel here")


# speedup vs baseline: 813.6838x; 813.6838x over previous
import functools

import jax
import jax.numpy as jnp
from jax.experimental import pallas as pl

WINDOW = 16
NUM_BINS = 100
SCALE = 0.1
EPS = 1e-08
INV_H = (NUM_BINS - 2) / (2.0 * SCALE)
MID = (NUM_BINS - 2) // 2


def _tok_kernel(vals_ref, bins_ref, lf_ref):
    x = vals_ref[0]
    tc, s = x.shape
    nw = tc // WINDOW
    w = x.reshape(nw, WINDOW, s)

    last = w[:, WINDOW - 1, :]
    mean = jnp.mean(w, axis=1)
    centered = w - mean[:, None, :]
    std = jnp.sqrt(jnp.mean(centered * centered, axis=1)) + EPS
    lf_ref[0] = jnp.stack([last, mean, std], axis=-1)

    prev = w[:, : WINDOW - 1, :]
    nxt = w[:, 1:, :]
    delta = (nxt - prev) / jnp.maximum(jnp.abs(prev), EPS)
    u = delta * INV_H + float(MID)
    b = jnp.clip(jnp.ceil(u), 0.0, float(NUM_BINS - 1)).astype(jnp.int32)
    bins_ref[0] = jnp.swapaxes(b, 1, 2)


@functools.partial(jax.jit, static_argnames=("tc",))
def _run(values, tc=512):
    bsz, t, s = values.shape
    nw_total = t // WINDOW
    nchunks = t // tc
    nw = tc // WINDOW
    bins, lf = pl.pallas_call(
        _tok_kernel,
        grid=(bsz, nchunks),
        in_specs=[pl.BlockSpec((1, tc, s), lambda b, c: (b, c, 0))],
        out_specs=[
            pl.BlockSpec((1, nw, s, WINDOW - 1), lambda b, c: (b, c, 0, 0)),
            pl.BlockSpec((1, nw, s, 3), lambda b, c: (b, c, 0, 0)),
        ],
        out_shape=[
            jax.ShapeDtypeStruct((bsz, nw_total, s, WINDOW - 1), jnp.int32),
            jax.ShapeDtypeStruct((bsz, nw_total, s, 3), jnp.float32),
        ],
    )(values)
    return bins, lf


def kernel(values):
    bins, lf = _run(values)
    return bins.astype(jnp.int64), lf


# speedup vs baseline: 1502.9525x; 1.8471x over previous
import functools

import jax
import jax.numpy as jnp
from jax.experimental import pallas as pl

WINDOW = 16
NUM_BINS = 100
SCALE = 0.1
EPS = 1e-08
INV_H = (NUM_BINS - 2) / (2.0 * SCALE)
MID = (NUM_BINS - 2) // 2


def _tok_kernel(vals_ref, bins_ref, last_ref, mean_ref, std_ref):
    x = vals_ref[0]
    tc, s = x.shape
    nw = tc // WINDOW
    w = x.reshape(nw, WINDOW, s)

    last_ref[0] = w[:, WINDOW - 1, :]
    mean = jnp.mean(w, axis=1)
    mean_ref[0] = mean
    centered = w - mean[:, None, :]
    std_ref[0] = jnp.sqrt(jnp.mean(centered * centered, axis=1)) + EPS

    prev = w[:, : WINDOW - 1, :]
    nxt = w[:, 1:, :]
    delta = (nxt - prev) / jnp.maximum(jnp.abs(prev), EPS)
    u = delta * INV_H + float(MID)
    b = jnp.clip(jnp.ceil(u), 0.0, float(NUM_BINS - 1)).astype(jnp.int32)
    bins_ref[0] = jnp.swapaxes(b, 1, 2)


@functools.partial(jax.jit, static_argnames=("tc",))
def _run(values, tc=1024):
    bsz, t, s = values.shape
    nw_total = t // WINDOW
    nchunks = t // tc
    nw = tc // WINDOW
    stat = pl.BlockSpec((1, nw, s), lambda b, c: (b, c, 0))
    stat_shape = jax.ShapeDtypeStruct((bsz, nw_total, s), jnp.float32)
    bins, last, mean, std = pl.pallas_call(
        _tok_kernel,
        grid=(bsz, nchunks),
        in_specs=[pl.BlockSpec((1, tc, s), lambda b, c: (b, c, 0))],
        out_specs=[
            pl.BlockSpec((1, nw, s, WINDOW - 1), lambda b, c: (b, c, 0, 0)),
            stat, stat, stat,
        ],
        out_shape=[
            jax.ShapeDtypeStruct((bsz, nw_total, s, WINDOW - 1), jnp.int32),
            stat_shape, stat_shape, stat_shape,
        ],
    )(values)
    return bins, last, mean, std


def kernel(values):
    bins, last, mean, std = _run(values)
    lf = jnp.stack([last, mean, std], axis=-1)
    return bins.astype(jnp.int64), lf


# speedup vs baseline: 1832.7223x; 1.2194x over previous
import functools

import jax
import jax.numpy as jnp
from jax.experimental import pallas as pl

WINDOW = 16
NUM_BINS = 100
SCALE = 0.1
EPS = 1e-08
INV_H = (NUM_BINS - 2) / (2.0 * SCALE)
MID = (NUM_BINS - 2) // 2


def _tok_kernel(vals_ref, bins_ref, last_ref, mean_ref, std_ref):
    x = vals_ref[0]
    tc, s = x.shape
    nw = tc // WINDOW
    w = x.reshape(nw, WINDOW, s)

    last_ref[0] = w[:, WINDOW - 1, :]
    mean = jnp.mean(w, axis=1)
    mean_ref[0] = mean
    centered = w - mean[:, None, :]
    std_ref[0] = jnp.sqrt(jnp.mean(centered * centered, axis=1)) + EPS

    prev = w[:, : WINDOW - 1, :]
    nxt = w[:, 1:, :]
    delta = (nxt - prev) / jnp.maximum(jnp.abs(prev), EPS)
    u = delta * INV_H + float(MID)
    b = jnp.clip(jnp.ceil(u), 0.0, float(NUM_BINS - 1)).astype(jnp.int32)
    bins_ref[0] = jnp.swapaxes(b, 1, 2).reshape(nw, s * (WINDOW - 1))


@functools.partial(jax.jit, static_argnames=("tc",))
def _run(values, tc=1024):
    bsz, t, s = values.shape
    nw_total = t // WINDOW
    nchunks = t // tc
    nw = tc // WINDOW
    stat = pl.BlockSpec((1, nw, s), lambda b, c: (b, c, 0))
    stat_shape = jax.ShapeDtypeStruct((bsz, nw_total, s), jnp.float32)
    bins, last, mean, std = pl.pallas_call(
        _tok_kernel,
        grid=(bsz, nchunks),
        in_specs=[pl.BlockSpec((1, tc, s), lambda b, c: (b, c, 0))],
        out_specs=[
            pl.BlockSpec((1, nw, s * (WINDOW - 1)), lambda b, c: (b, c, 0)),
            stat, stat, stat,
        ],
        out_shape=[
            jax.ShapeDtypeStruct((bsz, nw_total, s * (WINDOW - 1)), jnp.int32),
            stat_shape, stat_shape, stat_shape,
        ],
    )(values)
    return bins, last, mean, std


def kernel(values):
    bins, last, mean, std = _run(values)
    bsz, nw_total, _ = bins.shape
    lf = jnp.stack([last, mean, std], axis=-1)
    bins = bins.reshape(bsz, nw_total, lf.shape[2], WINDOW - 1)
    return bins.astype(jnp.int64), lf
